# baseline (speedup 1.0000x reference)
import functools

import jax
import jax.numpy as jnp
from jax import lax
from jax.experimental import pallas as pl
from jax.experimental.pallas import tpu as pltpu
from jax.experimental.pallas import tpu_sc as plsc

N = 16384
C = 1000
NC = 2
NS = 16
NW = NC * NS
ROWS_PER = N // NW
DMA_CH = 4
CH_W = ROWS_PER // DMA_CH

BR = 1024
G = N // BR


def _sc_alpha_gather(targets_2d, alpha_flat):
    mesh = plsc.VectorSubcoreMesh(core_axis_name="c", subcore_axis_name="s")

    @functools.partial(
        pl.kernel,
        mesh=mesh,
        out_type=jax.ShapeDtypeStruct((N // CH_W, CH_W), jnp.float32),
        scratch_types=[
            pltpu.VMEM((DMA_CH, CH_W), jnp.int32),
            pltpu.VMEM((DMA_CH, CH_W), jnp.float32),
            pltpu.VMEM_SHARED((C,), jnp.float32),
            pltpu.SemaphoreType.DMA,
        ],
    )
    def sc_kernel(tgt_hbm, alpha_hbm, out_hbm, t_v, a_v, alpha_s, sem):
        cid = lax.axis_index("c")
        sid = lax.axis_index("s")
        wid = sid * NC + cid

        @pl.when(sid == 0)
        def _stage():
            pltpu.sync_copy(alpha_hbm, alpha_s)

        pltpu.sync_copy(tgt_hbm.at[pl.ds(wid * DMA_CH, DMA_CH)], t_v)
        plsc.subcore_barrier()
        copies = [
            pltpu.async_copy(alpha_s.at[t_v.at[j]], a_v.at[j], sem)
            for j in range(DMA_CH)
        ]
        for cp in copies:
            cp.wait()
        pltpu.sync_copy(a_v, out_hbm.at[pl.ds(wid * DMA_CH, DMA_CH)])

    return sc_kernel(targets_2d, alpha_flat)


def _tc_g(inputs_t, targets_3d):
    def body(x_ref, t_ref, g_ref):
        xb = x_ref[...]
        t = t_ref[0]
        cls = lax.broadcasted_iota(jnp.int32, (C, BR), 0)
        xg = jnp.sum(jnp.where(cls == t, xb, 0.0), axis=0, keepdims=True)
        om = 1.0 - jnp.exp(xg)
        g_ref[0] = om * om * xg

    return pl.pallas_call(
        body,
        grid=(G,),
        in_specs=[
            pl.BlockSpec((C, BR), lambda j: (0, j)),
            pl.BlockSpec((1, 1, BR), lambda j: (j, 0, 0)),
        ],
        out_specs=pl.BlockSpec((1, 1, BR), lambda j: (j, 0, 0)),
        out_shape=jax.ShapeDtypeStruct((G, 1, BR), jnp.float32),
    )(inputs_t, targets_3d)


def _tc_combine(g3, a3):
    def body(g_ref, a_ref, o_ref):
        o_ref[0, 0] = -jnp.sum(g_ref[...] * a_ref[...]) * (1.0 / N)

    return pl.pallas_call(
        body,
        out_specs=pl.BlockSpec(memory_space=pltpu.SMEM),
        out_shape=jax.ShapeDtypeStruct((1, 1), jnp.float32),
    )(g3, a3)


def kernel(inputs, targets, alpha):
    tgt = targets.astype(jnp.int32)
    alpha_flat = alpha.reshape(-1).astype(jnp.float32)
    a_sel = _sc_alpha_gather(tgt.reshape(N // CH_W, CH_W), alpha_flat)
    g3 = _tc_g(inputs.T, tgt.reshape(G, 1, BR))
    out = _tc_combine(g3, a_sel.reshape(G, 1, BR))
    return out[0, 0]


# speedup vs baseline: 1.0152x; 1.0152x over previous
import functools

import jax
import jax.numpy as jnp
from jax import lax
from jax.experimental import pallas as pl
from jax.experimental.pallas import tpu as pltpu
from jax.experimental.pallas import tpu_sc as plsc

N = 16384
C = 1000
NC = 2
NS = 16
NW = NC * NS
ROWS_PER = N // NW
DMA_CH = 4
CH_W = ROWS_PER // DMA_CH

BR = 4096
G = N // BR


def _sc_alpha_gather(targets_2d, alpha_flat):
    mesh = plsc.VectorSubcoreMesh(core_axis_name="c", subcore_axis_name="s")

    @functools.partial(
        pl.kernel,
        mesh=mesh,
        out_type=jax.ShapeDtypeStruct((N // CH_W, CH_W), jnp.float32),
        scratch_types=[
            pltpu.VMEM((DMA_CH, CH_W), jnp.int32),
            pltpu.VMEM((DMA_CH, CH_W), jnp.float32),
            pltpu.VMEM_SHARED((C,), jnp.float32),
            pltpu.SemaphoreType.DMA,
        ],
    )
    def sc_kernel(tgt_hbm, alpha_hbm, out_hbm, t_v, a_v, alpha_s, sem):
        cid = lax.axis_index("c")
        sid = lax.axis_index("s")
        wid = sid * NC + cid

        @pl.when(sid == 0)
        def _stage():
            pltpu.sync_copy(alpha_hbm, alpha_s)

        pltpu.sync_copy(tgt_hbm.at[pl.ds(wid * DMA_CH, DMA_CH)], t_v)
        plsc.subcore_barrier()
        copies = [
            pltpu.async_copy(alpha_s.at[t_v.at[j]], a_v.at[j], sem)
            for j in range(DMA_CH)
        ]
        for cp in copies:
            cp.wait()
        pltpu.sync_copy(a_v, out_hbm.at[pl.ds(wid * DMA_CH, DMA_CH)])

    return sc_kernel(targets_2d, alpha_flat)


def _tc_g(inputs_t, targets_3d):
    def body(x_ref, t_ref, g_ref):
        xb = x_ref[...]
        t = t_ref[0]
        cls = lax.broadcasted_iota(jnp.int32, (C, BR), 0)
        xg = jnp.sum(jnp.where(cls == t, xb, 0.0), axis=0, keepdims=True)
        om = 1.0 - jnp.exp(xg)
        g_ref[0] = om * om * xg

    return pl.pallas_call(
        body,
        grid=(G,),
        in_specs=[
            pl.BlockSpec((C, BR), lambda j: (0, j)),
            pl.BlockSpec((1, 1, BR), lambda j: (j, 0, 0)),
        ],
        out_specs=pl.BlockSpec((1, 1, BR), lambda j: (j, 0, 0)),
        out_shape=jax.ShapeDtypeStruct((G, 1, BR), jnp.float32),
    )(inputs_t, targets_3d)


def _tc_combine(g3, a3):
    def body(g_ref, a_ref, o_ref):
        o_ref[0, 0] = -jnp.sum(g_ref[...] * a_ref[...]) * (1.0 / N)

    return pl.pallas_call(
        body,
        out_specs=pl.BlockSpec(memory_space=pltpu.SMEM),
        out_shape=jax.ShapeDtypeStruct((1, 1), jnp.float32),
    )(g3, a3)


def kernel(inputs, targets, alpha):
    tgt = targets.astype(jnp.int32)
    alpha_flat = alpha.reshape(-1).astype(jnp.float32)
    a_sel = _sc_alpha_gather(tgt.reshape(N // CH_W, CH_W), alpha_flat)
    g3 = _tc_g(inputs.T, tgt.reshape(G, 1, BR))
    out = _tc_combine(g3, a_sel.reshape(G, 1, BR))
    return out[0, 0]


# speedup vs baseline: 1.0626x; 1.0467x over previous
import functools

import jax
import jax.numpy as jnp
from jax import lax
from jax.experimental import pallas as pl
from jax.experimental.pallas import tpu as pltpu
from jax.experimental.pallas import tpu_sc as plsc

N = 16384
C = 1000
NC = 2
NS = 16
NW = NC * NS
ROWS_PER = N // NW
DMA_CH = 4
CH_W = ROWS_PER // DMA_CH

BR = 2048
G = N // BR


def _sc_alpha_gather(targets_2d, alpha_flat):
    mesh = plsc.VectorSubcoreMesh(core_axis_name="c", subcore_axis_name="s")

    @functools.partial(
        pl.kernel,
        mesh=mesh,
        out_type=jax.ShapeDtypeStruct((N // CH_W, CH_W), jnp.float32),
        scratch_types=[
            pltpu.VMEM((DMA_CH, CH_W), jnp.int32),
            pltpu.VMEM((DMA_CH, CH_W), jnp.float32),
            pltpu.VMEM_SHARED((C,), jnp.float32),
            pltpu.SemaphoreType.DMA,
        ],
    )
    def sc_kernel(tgt_hbm, alpha_hbm, out_hbm, t_v, a_v, alpha_s, sem):
        cid = lax.axis_index("c")
        sid = lax.axis_index("s")
        wid = sid * NC + cid

        @pl.when(sid == 0)
        def _stage():
            pltpu.sync_copy(alpha_hbm, alpha_s)

        pltpu.sync_copy(tgt_hbm.at[pl.ds(wid * DMA_CH, DMA_CH)], t_v)
        plsc.subcore_barrier()
        copies = [
            pltpu.async_copy(alpha_s.at[t_v.at[j]], a_v.at[j], sem)
            for j in range(DMA_CH)
        ]
        for cp in copies:
            cp.wait()
        pltpu.sync_copy(a_v, out_hbm.at[pl.ds(wid * DMA_CH, DMA_CH)])

    return sc_kernel(targets_2d, alpha_flat)


def _tc_g(inputs_t, targets_3d):
    def body(x_ref, t_ref, g_ref):
        xb = x_ref[...]
        t = t_ref[0]
        cls = lax.broadcasted_iota(jnp.int32, (C, BR), 0)
        xg = jnp.sum(jnp.where(cls == t, xb, 0.0), axis=0, keepdims=True)
        om = 1.0 - jnp.exp(xg)
        g_ref[0] = om * om * xg

    return pl.pallas_call(
        body,
        grid=(G,),
        in_specs=[
            pl.BlockSpec((C, BR), lambda j: (0, j)),
            pl.BlockSpec((1, 1, BR), lambda j: (j, 0, 0)),
        ],
        out_specs=pl.BlockSpec((1, 1, BR), lambda j: (j, 0, 0)),
        out_shape=jax.ShapeDtypeStruct((G, 1, BR), jnp.float32),
    )(inputs_t, targets_3d)


def _tc_combine(g3, a3):
    def body(g_ref, a_ref, o_ref):
        o_ref[0, 0] = -jnp.sum(g_ref[...] * a_ref[...]) * (1.0 / N)

    return pl.pallas_call(
        body,
        out_specs=pl.BlockSpec(memory_space=pltpu.SMEM),
        out_shape=jax.ShapeDtypeStruct((1, 1), jnp.float32),
    )(g3, a3)


def kernel(inputs, targets, alpha):
    tgt = targets.astype(jnp.int32)
    alpha_flat = alpha.reshape(-1).astype(jnp.float32)
    a_sel = _sc_alpha_gather(tgt.reshape(N // CH_W, CH_W), alpha_flat)
    g3 = _tc_g(inputs.T, tgt.reshape(G, 1, BR))
    out = _tc_combine(g3, a_sel.reshape(G, 1, BR))
    return out[0, 0]
